# baseline (speedup 1.0000x reference)
import functools

import jax
import jax.numpy as jnp
from jax.experimental import pallas as pl


def _sage_layer_body(apply_relu, ksplit, *refs):
    adj_refs = refs[:ksplit]
    xblk_ref, feats_ref, wa_ref, wb_ref, out_ref = refs[ksplit:]
    kw = feats_ref.shape[0] // ksplit
    p = None
    deg = None
    for j, aref in enumerate(adj_refs):
        a = aref[...]
        pj = jnp.dot(a.astype(jnp.bfloat16),
                     feats_ref[j * kw:(j + 1) * kw, :].astype(jnp.bfloat16),
                     preferred_element_type=jnp.float32)
        dj = jnp.sum(a, axis=1, keepdims=True)
        p = pj if p is None else p + pj
        deg = dj if deg is None else deg + dj
    neigh = p / (deg + 1.0)
    out = (jnp.dot(xblk_ref[...], wa_ref[...], preferred_element_type=jnp.float32)
           + jnp.dot(neigh, wb_ref[...], preferred_element_type=jnp.float32))
    if apply_relu:
        out = jnp.maximum(out, 0.0)
    out_ref[...] = out


def _make_adj_spec(bm, kw, j):
    return pl.BlockSpec((bm, kw), lambda i: (i, j))


def _sage_layer(adj, feats, wa, wb, apply_relu, bm, ksplit):
    n, d = feats.shape
    dh = wa.shape[1]
    kw = n // ksplit
    in_specs = [_make_adj_spec(bm, kw, j) for j in range(ksplit)]
    in_specs += [
        pl.BlockSpec((bm, d), lambda i: (i, 0)),
        pl.BlockSpec((n, d), lambda i: (0, 0)),
        pl.BlockSpec((d, dh), lambda i: (0, 0)),
        pl.BlockSpec((d, dh), lambda i: (0, 0)),
    ]
    return pl.pallas_call(
        functools.partial(_sage_layer_body, apply_relu, ksplit),
        grid=(n // bm,),
        in_specs=in_specs,
        out_specs=pl.BlockSpec((bm, dh), lambda i: (i, 0)),
        out_shape=jax.ShapeDtypeStruct((n, dh), jnp.float32),
    )(*([adj] * ksplit), feats, feats, wa, wb)


def kernel(adj, inputs, W0, W1):
    d_in = inputs.shape[1]
    wa0, wb0 = W0[:, :d_in].T, W0[:, d_in:].T
    h = _sage_layer(adj, inputs, wa0, wb0, apply_relu=True, bm=512, ksplit=4)
    dh = h.shape[1]
    wa1, wb1 = W1[:, :dh].T, W1[:, dh:].T
    return _sage_layer(adj, h, wa1, wb1, apply_relu=False, bm=512, ksplit=4)


# speedup vs baseline: 1.0391x; 1.0391x over previous
import functools

import jax
import jax.numpy as jnp
from jax.experimental import pallas as pl


def _layer0_body(adj_ref, xblk_ref, feats_ref, wa_ref, wb_ref,
                 h_ref, deg_ref):
    a = adj_ref[...]
    p = jnp.dot(a.astype(jnp.bfloat16), feats_ref[...],
                preferred_element_type=jnp.float32)
    deg = jnp.sum(a, axis=1, keepdims=True) + 1.0
    neigh = (p / deg).astype(jnp.bfloat16)
    out = (jnp.dot(xblk_ref[...], wa_ref[...], preferred_element_type=jnp.float32)
           + jnp.dot(neigh, wb_ref[...], preferred_element_type=jnp.float32))
    h_ref[...] = jnp.maximum(out, 0.0).astype(jnp.bfloat16)
    deg_ref[...] = deg


def _layer1_body(adj_ref, xblk_ref, feats_ref, wa_ref, wb_ref, deg_ref,
                 out_ref):
    a = adj_ref[...]
    p = jnp.dot(a.astype(jnp.bfloat16), feats_ref[...],
                preferred_element_type=jnp.float32)
    neigh = (p / deg_ref[...]).astype(jnp.bfloat16)
    out_ref[...] = (
        jnp.dot(xblk_ref[...], wa_ref[...], preferred_element_type=jnp.float32)
        + jnp.dot(neigh, wb_ref[...], preferred_element_type=jnp.float32))


def kernel(adj, inputs, W0, W1):
    n, d_in = inputs.shape
    dh = W0.shape[0]
    bm = 512
    x16 = inputs.astype(jnp.bfloat16)
    wa0, wb0 = W0[:, :d_in].T.astype(jnp.bfloat16), W0[:, d_in:].T.astype(jnp.bfloat16)
    wa1, wb1 = W1[:, :dh].T.astype(jnp.bfloat16), W1[:, dh:].T.astype(jnp.bfloat16)

    h, deg = pl.pallas_call(
        _layer0_body,
        grid=(n // bm,),
        in_specs=[
            pl.BlockSpec((bm, n), lambda i: (i, 0)),
            pl.BlockSpec((bm, d_in), lambda i: (i, 0)),
            pl.BlockSpec((n, d_in), lambda i: (0, 0)),
            pl.BlockSpec((d_in, dh), lambda i: (0, 0)),
            pl.BlockSpec((d_in, dh), lambda i: (0, 0)),
        ],
        out_specs=[
            pl.BlockSpec((bm, dh), lambda i: (i, 0)),
            pl.BlockSpec((bm, 1), lambda i: (i, 0)),
        ],
        out_shape=[
            jax.ShapeDtypeStruct((n, dh), jnp.bfloat16),
            jax.ShapeDtypeStruct((n, 1), jnp.float32),
        ],
    )(adj, x16, x16, wa0, wb0)

    return pl.pallas_call(
        _layer1_body,
        grid=(n // bm,),
        in_specs=[
            pl.BlockSpec((bm, n), lambda i: (i, 0)),
            pl.BlockSpec((bm, dh), lambda i: (i, 0)),
            pl.BlockSpec((n, dh), lambda i: (0, 0)),
            pl.BlockSpec((dh, dh), lambda i: (0, 0)),
            pl.BlockSpec((dh, dh), lambda i: (0, 0)),
            pl.BlockSpec((bm, 1), lambda i: (i, 0)),
        ],
        out_specs=pl.BlockSpec((bm, dh), lambda i: (i, 0)),
        out_shape=jax.ShapeDtypeStruct((n, dh), jnp.float32),
    )(adj, h, h, wa1, wb1, deg)
